# Initial kernel scaffold; baseline (speedup 1.0000x reference)
import jax
import jax.numpy as jnp
from jax.experimental import pallas as pl


def kernel(features, type_mask, mp_idx_u0, dst_u0, mp_idx_u1, dst_u1, target_idx_user, mp_idx_i0, dst_i0, mp_idx_i1, dst_i1, target_idx_item, attn1_u, attn2_u, fc1_u_W, fc1_u_b, fc2_u_W, attn1_i, attn2_i, fc1_i_W, fc1_i_b, fc2_i_W, fc_user_W, fc_user_b, fc_item_W, fc_item_b):
    raise NotImplementedError("write your pallas kernel here")


# speedup vs baseline: 1.1232x; 1.1232x over previous
import jax
import jax.numpy as jnp
from jax.experimental import pallas as pl

N = 30000
D = 64
H = 4
L = 3
E = 50000
T = 10000
AV = 32
OUT = 64
NMP = 2


def _elu(x):
    return jnp.where(x > 0, x, jnp.expm1(x))


def _metapath_fused(features, mp_idx, dst, attn1, attn2):
    edata = features[mp_idx]
    hidden = jnp.mean(edata, axis=1)
    center = edata[:, -1, :]
    a1 = center @ attn1.T
    a2 = hidden @ attn2.T
    a = a1 + a2
    a = jnp.where(a > 0, a, 0.01 * a)
    ae = jnp.exp(a)
    asum = jax.ops.segment_sum(ae, dst, num_segments=T)
    num = jax.ops.segment_sum(
        ae[:, :, None] * hidden[:, None, :], dst, num_segments=T)
    return num, asum


def _ctr_front(features, mp_idxs, dsts, attn1, attn2):
    outs = []
    for i in range(NMP):
        num, asum = _metapath_fused(features, mp_idxs[i], dsts[i],
                                    attn1[i], attn2[i])
        o = num / (asum[:, :, None] + 1e-9)
        outs.append(_elu(o).reshape(T, H * D))
    return outs


BT = 1000


def _score_kernel(o0u_ref, o1u_ref, o0i_ref, o1i_ref,
                  fc1u_ref, fc1ub_ref, fc1i_ref, fc1ib_ref,
                  acc_ref):
    @pl.when(pl.program_id(0) == 0)
    def _():
        acc_ref[...] = jnp.zeros_like(acc_ref)

    def srow(o, fc1, fc1b):
        t = jnp.tanh(jnp.dot(o, fc1, preferred_element_type=jnp.float32)
                     + fc1b)
        return jnp.sum(t, axis=0)

    r0 = srow(o0u_ref[...], fc1u_ref[...], fc1ub_ref[...])
    r1 = srow(o1u_ref[...], fc1u_ref[...], fc1ub_ref[...])
    r2 = srow(o0i_ref[...], fc1i_ref[...], fc1ib_ref[...])
    r3 = srow(o1i_ref[...], fc1i_ref[...], fc1ib_ref[...])
    acc_ref[...] += jnp.stack([r0, r1, r2, r3], axis=0)


def _combine_kernel(o0u_ref, o1u_ref, o0i_ref, o1i_ref, beta_ref,
                    fcuW_ref, fcub_ref, fciW_ref, fcib_ref,
                    lu_ref, li_ref, hu_ref, hi_ref):
    hu = beta_ref[0, 0] * o0u_ref[...] + beta_ref[0, 1] * o1u_ref[...]
    hi = beta_ref[0, 2] * o0i_ref[...] + beta_ref[0, 3] * o1i_ref[...]
    hu_ref[...] = hu
    hi_ref[...] = hi
    lu_ref[...] = jnp.dot(hu, fcuW_ref[...],
                          preferred_element_type=jnp.float32) + fcub_ref[...]
    li_ref[...] = jnp.dot(hi, fciW_ref[...],
                          preferred_element_type=jnp.float32) + fcib_ref[...]


def kernel(features, type_mask, mp_idx_u0, dst_u0, mp_idx_u1, dst_u1,
           target_idx_user, mp_idx_i0, dst_i0, mp_idx_i1, dst_i1,
           target_idx_item,
           attn1_u, attn2_u, fc1_u_W, fc1_u_b, fc2_u_W,
           attn1_i, attn2_i, fc1_i_W, fc1_i_b, fc2_i_W,
           fc_user_W, fc_user_b, fc_item_W, fc_item_b):
    o0u, o1u = _ctr_front(features, [mp_idx_u0, mp_idx_u1],
                          [dst_u0, dst_u1], attn1_u, attn2_u)
    o0i, o1i = _ctr_front(features, [mp_idx_i0, mp_idx_i1],
                          [dst_i0, dst_i1], attn1_i, attn2_i)

    f32 = jnp.float32
    nb = T // BT
    ospec = pl.BlockSpec((BT, H * D), lambda i: (i, 0))
    wspec = pl.BlockSpec((H * D, AV), lambda i: (0, 0))
    bspec = pl.BlockSpec((AV,), lambda i: (0,))

    acc = pl.pallas_call(
        _score_kernel,
        grid=(nb,),
        in_specs=[ospec, ospec, ospec, ospec, wspec, bspec, wspec, bspec],
        out_specs=pl.BlockSpec((4, AV), lambda i: (0, 0)),
        out_shape=jax.ShapeDtypeStruct((4, AV), f32),
    )(o0u, o1u, o0i, o1i, fc1_u_W, fc1_u_b, fc1_i_W, fc1_i_b)

    s = jnp.sum(acc * jnp.stack([fc2_u_W, fc2_u_W, fc2_i_W, fc2_i_W]),
                axis=1) * (1.0 / T)
    bu = jax.nn.softmax(s[:2])
    bi = jax.nn.softmax(s[2:])
    beta = jnp.concatenate([bu, bi]).reshape(1, 4)

    lu, li, hu, hi = pl.pallas_call(
        _combine_kernel,
        grid=(nb,),
        in_specs=[ospec, ospec, ospec, ospec,
                  pl.BlockSpec((1, 4), lambda i: (0, 0)),
                  pl.BlockSpec((H * D, OUT), lambda i: (0, 0)),
                  pl.BlockSpec((OUT,), lambda i: (0,)),
                  pl.BlockSpec((H * D, OUT), lambda i: (0, 0)),
                  pl.BlockSpec((OUT,), lambda i: (0,))],
        out_specs=(pl.BlockSpec((BT, OUT), lambda i: (i, 0)),
                   pl.BlockSpec((BT, OUT), lambda i: (i, 0)),
                   pl.BlockSpec((BT, H * D), lambda i: (i, 0)),
                   pl.BlockSpec((BT, H * D), lambda i: (i, 0))),
        out_shape=(
            jax.ShapeDtypeStruct((T, OUT), f32),
            jax.ShapeDtypeStruct((T, OUT), f32),
            jax.ShapeDtypeStruct((T, H * D), f32),
            jax.ShapeDtypeStruct((T, H * D), f32),
        ),
    )(o0u, o1u, o0i, o1i, beta,
      fc_user_W, fc_user_b, fc_item_W, fc_item_b)
    return (lu, li, hu, hi)


# speedup vs baseline: 31.5552x; 28.0940x over previous
import functools

import jax
import jax.numpy as jnp
from jax import lax
from jax.experimental import pallas as pl
from jax.experimental.pallas import tpu as pltpu
from jax.experimental.pallas import tpu_sc as plsc

N = 30000
D = 64
H = 4
L = 3
E = 50000
T = 10000
AV = 32
OUT = 64
NMP = 2

NTILE = 32
TPM = 8
TRANGE = T // TPM
BLK = 384
WROW = 125
IPAD = 151296
DPAD = 50432

_THIRD = 1.0 / 3.0


def _bcast_i32(x):
    return jnp.broadcast_to(jnp.asarray(x, jnp.int32), (16,))


_GDN = lax.GatherDimensionNumbers(
    offset_dims=(), collapsed_slice_dims=(0,), start_index_map=(0,))


def _vperm(v, idx):
    return lax.gather(v, idx[:, None], _GDN, (1,),
                      mode=lax.GatherScatterMode.PROMISE_IN_BOUNDS)


def _hsum(v):
    ii = lax.iota(jnp.int32, 16)
    for sh in (8, 4, 2, 1):
        v = v + _vperm(v, (ii + sh) % 16)
    return v


def _sget(ref, idx):
    return ref[pl.ds(idx, 16)][0]


def _sput(ref, pos, val):
    slot = (pos // 16) * 16
    lane = pos - slot
    v = ref[pl.ds(slot, 16)]
    ref[pl.ds(slot, 16)] = jnp.where(
        lax.iota(jnp.int32, 16) == lane, _bcast_i32(val), v)


def _sc_front_body(features, idxp, dstp, starts, wpack, o_raw,
                   idxwin, rowsbuf, dstblk, outwin, wbuf,
                   startbuf, smem, gsem):
    c = lax.axis_index("c")
    s = lax.axis_index("s")
    tile = c * 16 + s
    m = tile // TPM
    j = tile % TPM

    pltpu.sync_copy(starts.at[pl.ds(m * 16, 16)], startbuf.at[pl.ds(0, 16)])
    pltpu.sync_copy(wpack.at[pl.ds(m * 512, 512)], wbuf)

    e0 = _sget(startbuf, j)
    e1 = _sget(startbuf, j + 1)
    t_lo = j * TRANGE
    t_hi = t_lo + TRANGE
    eb0 = (e0 // BLK) * BLK
    cnt = e1 - eb0
    nblk = (cnt + BLK - 1) // BLK

    w = [[wbuf[pl.ds(r * 64 + 16 * k, 16)] for k in range(4)]
         for r in range(8)]

    zero_v = jnp.zeros((16,), jnp.float32)
    smem[0] = -1
    smem[1] = t_lo
    smem[2] = t_lo

    def write_row(t, rowvecs):
        off = (t - smem[1]) * 256
        for q in range(16):
            outwin[pl.ds(off + 16 * q, 16)] = rowvecs[q]

    def zero_upto(tend):
        def zb(t, _):
            write_row(t, [zero_v] * 16)
            return 0
        lax.fori_loop(smem[2], tend, zb, 0)
        smem[2] = tend

    def flush_window():
        w0 = smem[1]
        zero_upto(w0 + WROW)
        pltpu.sync_copy(outwin,
                        o_raw.at[pl.ds((m * T + w0) * 256, WROW * 256)])
        smem[1] = w0 + WROW
        smem[2] = w0 + WROW

    def advance_to(d):
        nfl = (d - smem[1]) // WROW

        def fb(t, _):
            flush_window()
            return 0
        lax.fori_loop(0, nfl, fb, 0)
        zero_upto(d)

    def flush_seg(cur_d, asumv, acc):
        rows = []
        for h in range(4):
            rec = 1.0 / (asumv[h] + 1e-9)
            for k in range(4):
                rows.append(acc[4 * h + k] * rec)
        write_row(cur_d, rows)
        smem[2] = cur_d + 1

    def block_body(b, st):
        base = eb0 + b * BLK
        pltpu.sync_copy(dstp.at[pl.ds(m * DPAD + base, BLK)],
                        dstblk.at[pl.ds(0, BLK)])
        pltpu.sync_copy(idxp.at[pl.ds(m * IPAD + 3 * base, 3 * BLK)], idxwin)
        handles = []
        for r in range(9):
            handles.append(pltpu.async_copy(
                features.at[idxwin.at[pl.ds(r * 128, 128)]],
                rowsbuf.at[pl.ds(r * 128, 128)], gsem))
        for hnd in handles:
            hnd.wait()

        ecnt = jnp.minimum(BLK, cnt - b * BLK)

        def edge_body(i, st2):
            cur_d = st2[0]
            asumv = list(st2[1:5])
            acc = list(st2[5:21])
            d = _sget(dstblk, i)
            r0 = [rowsbuf[3 * i, pl.ds(16 * k, 16)] for k in range(4)]
            r1 = [rowsbuf[3 * i + 1, pl.ds(16 * k, 16)] for k in range(4)]
            r2 = [rowsbuf[3 * i + 2, pl.ds(16 * k, 16)] for k in range(4)]
            s01 = [r0[k] + r1[k] for k in range(4)]
            hid = [(s01[k] + r2[k]) * _THIRD for k in range(4)]
            active = d >= t_lo
            gate = jnp.where(active, 1.0, 0.0)
            aev = []
            for h in range(4):
                t = r2[0] * w[h][0]
                for k in range(1, 4):
                    t = t + r2[k] * w[h][k]
                for k in range(4):
                    t = t + s01[k] * w[4 + h][k]
                tot = _hsum(t)
                a = jnp.where(tot > 0, tot, 0.01 * tot)
                aev.append(jnp.exp(a) * gate)

            is_new = active & (d != cur_d)

            @pl.when(is_new)
            def _():
                @pl.when(cur_d >= 0)
                def _():
                    flush_seg(cur_d, asumv, acc)
                advance_to(d)
                smem[0] = d

            keep = jnp.where(is_new, 0.0, 1.0)
            new_asum = [asumv[h] * keep + aev[h] for h in range(4)]
            new_acc = [acc[4 * h + k] * keep + aev[h] * hid[k]
                       for h in range(4) for k in range(4)]
            cur_d2 = jnp.where(is_new, d, cur_d)
            return (cur_d2, *new_asum, *new_acc)

        return lax.fori_loop(0, ecnt, edge_body, st)

    st0 = (jnp.int32(-1), *([zero_v] * 20))
    stF = lax.fori_loop(0, nblk, block_body, st0)

    cur_d = stF[0]

    @pl.when(cur_d >= 0)
    def _():
        flush_seg(cur_d, list(stF[1:5]), list(stF[5:21]))
    advance_to(t_hi)


def _elu(x):
    return jnp.where(x > 0, x, jnp.exp(jnp.minimum(x, 0.0)) - 1.0)


BT = 1000


def _score_kernel(o0u_ref, o1u_ref, o0i_ref, o1i_ref,
                  fc1u_ref, fc1ub_ref, fc1i_ref, fc1ib_ref,
                  acc_ref):
    @pl.when(pl.program_id(0) == 0)
    def _():
        acc_ref[...] = jnp.zeros_like(acc_ref)

    def srow(o, fc1, fc1b):
        t = jnp.tanh(jnp.dot(_elu(o), fc1,
                             preferred_element_type=jnp.float32) + fc1b)
        return jnp.sum(t, axis=0)

    r0 = srow(o0u_ref[...], fc1u_ref[...], fc1ub_ref[...])
    r1 = srow(o1u_ref[...], fc1u_ref[...], fc1ub_ref[...])
    r2 = srow(o0i_ref[...], fc1i_ref[...], fc1ib_ref[...])
    r3 = srow(o1i_ref[...], fc1i_ref[...], fc1ib_ref[...])
    acc_ref[...] += jnp.stack([r0, r1, r2, r3], axis=0)


def _combine_kernel(o0u_ref, o1u_ref, o0i_ref, o1i_ref, beta_ref,
                    fcuW_ref, fcub_ref, fciW_ref, fcib_ref,
                    lu_ref, li_ref, hu_ref, hi_ref):
    hu = (beta_ref[0, 0] * _elu(o0u_ref[...])
          + beta_ref[0, 1] * _elu(o1u_ref[...]))
    hi = (beta_ref[0, 2] * _elu(o0i_ref[...])
          + beta_ref[0, 3] * _elu(o1i_ref[...]))
    hu_ref[...] = hu
    hi_ref[...] = hi
    lu_ref[...] = jnp.dot(hu, fcuW_ref[...],
                          preferred_element_type=jnp.float32) + fcub_ref[...]
    li_ref[...] = jnp.dot(hi, fciW_ref[...],
                          preferred_element_type=jnp.float32) + fcib_ref[...]


def kernel(features, type_mask, mp_idx_u0, dst_u0, mp_idx_u1, dst_u1,
           target_idx_user, mp_idx_i0, dst_i0, mp_idx_i1, dst_i1,
           target_idx_item,
           attn1_u, attn2_u, fc1_u_W, fc1_u_b, fc2_u_W,
           attn1_i, attn2_i, fc1_i_W, fc1_i_b, fc2_i_W,
           fc_user_W, fc_user_b, fc_item_W, fc_item_b):
    f32 = jnp.float32
    i32 = jnp.int32

    mp_idxs = [mp_idx_u0, mp_idx_u1, mp_idx_i0, mp_idx_i1]
    dsts = [dst_u0, dst_u1, dst_i0, dst_i1]
    attn1s = [attn1_u[0], attn1_u[1], attn1_i[0], attn1_i[1]]
    attn2s = [attn2_u[0], attn2_u[1], attn2_i[0], attn2_i[1]]

    idx_rows = []
    dst_rows = []
    starts_rows = []
    for mi in range(4):
        flat = mp_idxs[mi].reshape(-1).astype(i32)
        idx_rows.append(jnp.concatenate(
            [flat, jnp.zeros((IPAD - 3 * E,), i32)]))
        dd = dsts[mi].astype(i32)
        dst_rows.append(jnp.concatenate(
            [dd, jnp.zeros((DPAD - E,), i32)]))
        bounds = jnp.arange(TPM + 1, dtype=i32) * TRANGE
        st = jnp.searchsorted(dd, bounds, side="left").astype(i32)
        starts_rows.append(jnp.concatenate(
            [st, jnp.zeros((16 - TPM - 1,), i32)]))
    idxp = jnp.concatenate(idx_rows)
    dstp = jnp.concatenate(dst_rows)
    starts = jnp.concatenate(starts_rows)

    wpk = []
    for mi in range(4):
        w1 = attn1s[mi] + attn2s[mi] * _THIRD
        w2 = attn2s[mi] * _THIRD
        wpk.append(jnp.concatenate([w1, w2], axis=0))
    wpack = jnp.concatenate([x.reshape(-1) for x in wpk])

    mesh = plsc.VectorSubcoreMesh(core_axis_name="c", subcore_axis_name="s")
    sc_front = functools.partial(
        pl.kernel, mesh=mesh,
        compiler_params=pltpu.CompilerParams(use_tc_tiling_on_sc=False),
        out_type=jax.ShapeDtypeStruct((4 * T * 256,), f32),
        scratch_types=[
            pltpu.VMEM((3 * BLK,), i32),
            pltpu.VMEM((3 * BLK, 64), f32),
            pltpu.VMEM((BLK + 128,), i32),
            pltpu.VMEM((WROW * 256,), f32),
            pltpu.VMEM((512,), f32),
            pltpu.VMEM((128,), i32),
            pltpu.SMEM((4,), i32),
            pltpu.SemaphoreType.DMA,
        ],
    )(_sc_front_body)
    o_raw = sc_front(features, idxp, dstp, starts, wpack)
    o_raw = o_raw.reshape(4 * T, 256)

    nb = T // BT

    def ospec(mi):
        return pl.BlockSpec((BT, 256), lambda i, mi=mi: (mi * nb + i, 0))

    wspec = pl.BlockSpec((H * D, AV), lambda i: (0, 0))
    bspec = pl.BlockSpec((AV,), lambda i: (0,))

    acc = pl.pallas_call(
        _score_kernel,
        grid=(nb,),
        in_specs=[ospec(0), ospec(1), ospec(2), ospec(3),
                  wspec, bspec, wspec, bspec],
        out_specs=pl.BlockSpec((4, AV), lambda i: (0, 0)),
        out_shape=jax.ShapeDtypeStruct((4, AV), f32),
    )(o_raw, o_raw, o_raw, o_raw, fc1_u_W, fc1_u_b, fc1_i_W, fc1_i_b)

    s = jnp.sum(acc * jnp.stack([fc2_u_W, fc2_u_W, fc2_i_W, fc2_i_W]),
                axis=1) * (1.0 / T)
    bu = jax.nn.softmax(s[:2])
    bi = jax.nn.softmax(s[2:])
    beta = jnp.concatenate([bu, bi]).reshape(1, 4)

    lu, li, hu, hi = pl.pallas_call(
        _combine_kernel,
        grid=(nb,),
        in_specs=[ospec(0), ospec(1), ospec(2), ospec(3),
                  pl.BlockSpec((1, 4), lambda i: (0, 0)),
                  pl.BlockSpec((H * D, OUT), lambda i: (0, 0)),
                  pl.BlockSpec((OUT,), lambda i: (0,)),
                  pl.BlockSpec((H * D, OUT), lambda i: (0, 0)),
                  pl.BlockSpec((OUT,), lambda i: (0,))],
        out_specs=(pl.BlockSpec((BT, OUT), lambda i: (i, 0)),
                   pl.BlockSpec((BT, OUT), lambda i: (i, 0)),
                   pl.BlockSpec((BT, H * D), lambda i: (i, 0)),
                   pl.BlockSpec((BT, H * D), lambda i: (i, 0))),
        out_shape=(
            jax.ShapeDtypeStruct((T, OUT), f32),
            jax.ShapeDtypeStruct((T, OUT), f32),
            jax.ShapeDtypeStruct((T, H * D), f32),
            jax.ShapeDtypeStruct((T, H * D), f32),
        ),
    )(o_raw, o_raw, o_raw, o_raw, beta,
      fc_user_W, fc_user_b, fc_item_W, fc_item_b)
    return (lu, li, hu, hi)


# speedup vs baseline: 40.4366x; 1.2815x over previous
import functools

import jax
import jax.numpy as jnp
from jax import lax
from jax.experimental import pallas as pl
from jax.experimental.pallas import tpu as pltpu
from jax.experimental.pallas import tpu_sc as plsc

N = 30000
D = 64
H = 4
L = 3
E = 50000
T = 10000
AV = 32
OUT = 64
NMP = 2

NTILE = 32
TPM = 8
TRANGE = T // TPM
BLK = 128
WROW = 125
IPAD = 151296
DPAD = 50432

_THIRD = 1.0 / 3.0


def _bcast_i32(x):
    return jnp.broadcast_to(jnp.asarray(x, jnp.int32), (16,))


_GDN = lax.GatherDimensionNumbers(
    offset_dims=(), collapsed_slice_dims=(0,), start_index_map=(0,))


def _vperm(v, idx):
    return lax.gather(v, idx[:, None], _GDN, (1,),
                      mode=lax.GatherScatterMode.PROMISE_IN_BOUNDS)


def _hsum(v):
    ii = lax.iota(jnp.int32, 16)
    for sh in (8, 4, 2, 1):
        v = v + _vperm(v, (ii + sh) % 16)
    return v


def _sget(ref, idx):
    return ref[pl.ds(idx, 16)][0]


def _sput(ref, pos, val):
    slot = (pos // 16) * 16
    lane = pos - slot
    v = ref[pl.ds(slot, 16)]
    ref[pl.ds(slot, 16)] = jnp.where(
        lax.iota(jnp.int32, 16) == lane, _bcast_i32(val), v)


def _sc_front_body(features, idxp, dstp, starts, wpack, o_raw,
                   idxwin, rowsbuf, dstblk, outwin, wbuf,
                   startbuf, smem, isem0, isem1, gsem0, gsem1):
    c = lax.axis_index("c")
    s = lax.axis_index("s")
    tile = c * 16 + s
    m = tile // TPM
    j = tile % TPM

    pltpu.sync_copy(starts.at[pl.ds(m * 16, 16)], startbuf.at[pl.ds(0, 16)])
    pltpu.sync_copy(wpack.at[pl.ds(m * 512, 512)], wbuf)

    e0 = _sget(startbuf, j)
    e1 = _sget(startbuf, j + 1)
    t_lo = j * TRANGE
    t_hi = t_lo + TRANGE
    eb0 = (e0 // BLK) * BLK
    cnt = e1 - eb0
    nblk = (cnt + BLK - 1) // BLK

    w = [[wbuf[pl.ds(r * 64 + 16 * k, 16)] for k in range(4)]
         for r in range(8)]

    zero_v = jnp.zeros((16,), jnp.float32)
    smem[0] = -1
    smem[1] = t_lo
    smem[2] = t_lo

    def write_row(t, rowvecs):
        off = (t - smem[1]) * 256
        for q in range(16):
            outwin[pl.ds(off + 16 * q, 16)] = rowvecs[q]

    def zero_upto(tend):
        def zb(t, _):
            write_row(t, [zero_v] * 16)
            return 0
        lax.fori_loop(smem[2], tend, zb, 0)
        smem[2] = tend

    def flush_window():
        w0 = smem[1]
        zero_upto(w0 + WROW)
        pltpu.sync_copy(outwin,
                        o_raw.at[pl.ds((m * T + w0) * 256, WROW * 256)])
        smem[1] = w0 + WROW
        smem[2] = w0 + WROW

    def advance_to(d):
        nfl = (d - smem[1]) // WROW

        def fb(t, _):
            flush_window()
            return 0
        lax.fori_loop(0, nfl, fb, 0)
        zero_upto(d)

    def flush_seg(cur_d, asumv, acc):
        rows = []
        for h in range(4):
            rec = 1.0 / (asumv[h] + 1e-9)
            for k in range(4):
                rows.append(acc[4 * h + k] * rec)
        write_row(cur_d, rows)
        smem[2] = cur_d + 1

    isems = [isem0, isem1]
    gsems = [gsem0, gsem1]

    def fire_win(b, s2, sd):
        base = eb0 + b * BLK
        pltpu.async_copy(dstp.at[pl.ds(m * DPAD + base, BLK)],
                         dstblk.at[sd, pl.ds(0, BLK)], isems[s2])
        pltpu.async_copy(idxp.at[pl.ds(m * IPAD + 3 * base, 3 * BLK)],
                         idxwin.at[s2], isems[s2])

    def wait_win(b, s2, sd):
        base = eb0 + b * BLK
        pltpu.make_async_copy(dstp.at[pl.ds(m * DPAD + base, BLK)],
                              dstblk.at[sd, pl.ds(0, BLK)],
                              isems[s2]).wait()
        pltpu.make_async_copy(idxp.at[pl.ds(m * IPAD + 3 * base, 3 * BLK)],
                              idxwin.at[s2], isems[s2]).wait()

    def fire_gathers(s2):
        for r in range(3):
            pltpu.async_copy(
                features.at[idxwin.at[s2, pl.ds(r * 128, 128)]],
                rowsbuf.at[s2, pl.ds(r * 128, 128)], gsems[s2])

    def wait_gathers(s2):
        for r in range(3):
            pltpu.make_async_copy(
                features.at[idxwin.at[s2, pl.ds(r * 128, 128)]],
                rowsbuf.at[s2, pl.ds(r * 128, 128)], gsems[s2]).wait()

    @pl.when(nblk > 0)
    def _():
        fire_win(0, 0, 0)
        wait_win(0, 0, 0)
        fire_gathers(0)

    @pl.when(nblk > 1)
    def _():
        fire_win(1, 1, 1)

    def process_block(b, s2, sd, st):
        @pl.when(b + 1 < nblk)
        def _():
            wait_win(b + 1, s2 ^ 1, (sd + 1) % 4)
            fire_gathers(s2 ^ 1)

        @pl.when(b < nblk)
        def _():
            wait_gathers(s2)

        @pl.when(b + 2 < nblk)
        def _():
            fire_win(b + 2, s2, (sd + 2) % 4)

        base = eb0 + b * BLK
        ecnt = jnp.clip(cnt - b * BLK, 0, BLK)
        lo = jnp.clip(e0 - base, 0, BLK)

        def edge_body(i, st2):
            cur_d = st2[0]
            asumv = list(st2[1:5])
            acc = list(st2[5:21])
            d = _sget(dstblk.at[sd], i)
            r0 = [rowsbuf[s2, 3 * i, pl.ds(16 * k, 16)] for k in range(4)]
            r1 = [rowsbuf[s2, 3 * i + 1, pl.ds(16 * k, 16)]
                  for k in range(4)]
            r2 = [rowsbuf[s2, 3 * i + 2, pl.ds(16 * k, 16)]
                  for k in range(4)]
            s01 = [r0[k] + r1[k] for k in range(4)]
            hid = [(s01[k] + r2[k]) * _THIRD for k in range(4)]
            aev = []
            for h in range(4):
                t = r2[0] * w[h][0]
                for k in range(1, 4):
                    t = t + r2[k] * w[h][k]
                for k in range(4):
                    t = t + s01[k] * w[4 + h][k]
                tot = _hsum(t)
                a = jnp.where(tot > 0, tot, 0.01 * tot)
                aev.append(jnp.exp(a))

            is_new = d != cur_d

            @pl.when(is_new)
            def _():
                @pl.when(cur_d >= 0)
                def _():
                    flush_seg(cur_d, asumv, acc)
                advance_to(d)
                smem[0] = d

            keep = jnp.where(is_new, 0.0, 1.0)
            new_asum = [asumv[h] * keep + aev[h] for h in range(4)]
            new_acc = [acc[4 * h + k] * keep + aev[h] * hid[k]
                       for h in range(4) for k in range(4)]
            cur_d2 = jnp.where(is_new, d, cur_d)
            return (cur_d2, *new_asum, *new_acc)

        return lax.fori_loop(lo, jnp.maximum(ecnt, lo), edge_body, st2_init(st))

    def st2_init(st):
        return st

    def quad_body(p, st):
        for par in range(4):
            st = process_block(4 * p + par, par & 1, par, st)
        return st

    st0 = (jnp.int32(-1), *([zero_v] * 20))
    nquad = (nblk + 3) // 4
    stF = lax.fori_loop(0, nquad, quad_body, st0)

    cur_d = stF[0]

    @pl.when(cur_d >= 0)
    def _():
        flush_seg(cur_d, list(stF[1:5]), list(stF[5:21]))
    advance_to(t_hi)


def _elu(x):
    return jnp.where(x > 0, x, jnp.exp(jnp.minimum(x, 0.0)) - 1.0)


BT = 1000


def _back_kernel(o0u_ref, o1u_ref, o0i_ref, o1i_ref,
                 fc1u_ref, fc1ub_ref, fc1i_ref, fc1ib_ref,
                 f2u_ref, f2i_ref,
                 fcuW_ref, fcub_ref, fciW_ref, fcib_ref,
                 lu_ref, li_ref, hu_ref, hi_ref,
                 acc_ref, beta_ref):
    p = pl.program_id(0)
    i = pl.program_id(1)

    @pl.when((p == 0) & (i == 0))
    def _():
        acc_ref[...] = jnp.zeros_like(acc_ref)

    @pl.when(p == 0)
    def _():
        def srow(o, fc1, fc1b):
            t = jnp.tanh(jnp.dot(_elu(o), fc1,
                                 preferred_element_type=jnp.float32) + fc1b)
            return jnp.sum(t, axis=0)

        r0 = srow(o0u_ref[...], fc1u_ref[...], fc1ub_ref[...])
        r1 = srow(o1u_ref[...], fc1u_ref[...], fc1ub_ref[...])
        r2 = srow(o0i_ref[...], fc1i_ref[...], fc1ib_ref[...])
        r3 = srow(o1i_ref[...], fc1i_ref[...], fc1ib_ref[...])
        acc_ref[...] += jnp.stack([r0, r1, r2, r3], axis=0)

    @pl.when(p == 1)
    def _():
        @pl.when(i == 0)
        def _():
            s0 = jnp.sum(acc_ref[0, :] * f2u_ref[...]) * (1.0 / T)
            s1 = jnp.sum(acc_ref[1, :] * f2u_ref[...]) * (1.0 / T)
            s2 = jnp.sum(acc_ref[2, :] * f2i_ref[...]) * (1.0 / T)
            s3 = jnp.sum(acc_ref[3, :] * f2i_ref[...]) * (1.0 / T)
            mu = jnp.maximum(s0, s1)
            e0 = jnp.exp(s0 - mu)
            e1 = jnp.exp(s1 - mu)
            mi = jnp.maximum(s2, s3)
            e2 = jnp.exp(s2 - mi)
            e3 = jnp.exp(s3 - mi)
            beta_ref[0] = e0 / (e0 + e1)
            beta_ref[1] = e1 / (e0 + e1)
            beta_ref[2] = e2 / (e2 + e3)
            beta_ref[3] = e3 / (e2 + e3)

        hu = (beta_ref[0] * _elu(o0u_ref[...])
              + beta_ref[1] * _elu(o1u_ref[...]))
        hi = (beta_ref[2] * _elu(o0i_ref[...])
              + beta_ref[3] * _elu(o1i_ref[...]))
        hu_ref[...] = hu
        hi_ref[...] = hi
        lu_ref[...] = (jnp.dot(hu, fcuW_ref[...],
                               preferred_element_type=jnp.float32)
                       + fcub_ref[...])
        li_ref[...] = (jnp.dot(hi, fciW_ref[...],
                               preferred_element_type=jnp.float32)
                       + fcib_ref[...])


def kernel(features, type_mask, mp_idx_u0, dst_u0, mp_idx_u1, dst_u1,
           target_idx_user, mp_idx_i0, dst_i0, mp_idx_i1, dst_i1,
           target_idx_item,
           attn1_u, attn2_u, fc1_u_W, fc1_u_b, fc2_u_W,
           attn1_i, attn2_i, fc1_i_W, fc1_i_b, fc2_i_W,
           fc_user_W, fc_user_b, fc_item_W, fc_item_b):
    f32 = jnp.float32
    i32 = jnp.int32

    mp_idxs = [mp_idx_u0, mp_idx_u1, mp_idx_i0, mp_idx_i1]
    dsts = [dst_u0, dst_u1, dst_i0, dst_i1]
    attn1s = [attn1_u[0], attn1_u[1], attn1_i[0], attn1_i[1]]
    attn2s = [attn2_u[0], attn2_u[1], attn2_i[0], attn2_i[1]]

    idx_rows = []
    dst_rows = []
    starts_rows = []
    for mi in range(4):
        flat = mp_idxs[mi].reshape(-1).astype(i32)
        idx_rows.append(jnp.concatenate(
            [flat, jnp.zeros((IPAD - 3 * E,), i32)]))
        dd = dsts[mi].astype(i32)
        dst_rows.append(jnp.concatenate(
            [dd, jnp.zeros((DPAD - E,), i32)]))
        bounds = jnp.arange(TPM + 1, dtype=i32) * TRANGE
        st = jnp.sum(dd[None, :] < bounds[:, None], axis=1,
                     dtype=i32)
        starts_rows.append(jnp.concatenate(
            [st, jnp.zeros((16 - TPM - 1,), i32)]))
    idxp = jnp.concatenate(idx_rows)
    dstp = jnp.concatenate(dst_rows)
    starts = jnp.concatenate(starts_rows)

    wpk = []
    for mi in range(4):
        w1 = attn1s[mi] + attn2s[mi] * _THIRD
        w2 = attn2s[mi] * _THIRD
        wpk.append(jnp.concatenate([w1, w2], axis=0))
    wpack = jnp.concatenate([x.reshape(-1) for x in wpk])

    mesh = plsc.VectorSubcoreMesh(core_axis_name="c", subcore_axis_name="s")
    sc_front = functools.partial(
        pl.kernel, mesh=mesh,
        compiler_params=pltpu.CompilerParams(use_tc_tiling_on_sc=False),
        out_type=jax.ShapeDtypeStruct((4 * T * 256,), f32),
        scratch_types=[
            pltpu.VMEM((2, 3 * BLK), i32),
            pltpu.VMEM((2, 3 * BLK, 64), f32),
            pltpu.VMEM((4, BLK + 128), i32),
            pltpu.VMEM((WROW * 256,), f32),
            pltpu.VMEM((512,), f32),
            pltpu.VMEM((128,), i32),
            pltpu.SMEM((4,), i32),
            pltpu.SemaphoreType.DMA,
            pltpu.SemaphoreType.DMA,
            pltpu.SemaphoreType.DMA,
            pltpu.SemaphoreType.DMA,
        ],
    )(_sc_front_body)
    o_raw = sc_front(features, idxp, dstp, starts, wpack)
    o_raw = o_raw.reshape(4 * T, 256)

    nb = T // BT

    def ospec(mi):
        return pl.BlockSpec((BT, 256), lambda p, i, mi=mi: (mi * nb + i, 0))

    def cspec(shape):
        nd = len(shape)
        return pl.BlockSpec(shape, lambda p, i, nd=nd: (0,) * nd)

    lu, li, hu, hi = pl.pallas_call(
        _back_kernel,
        grid=(2, nb),
        in_specs=[ospec(0), ospec(1), ospec(2), ospec(3),
                  cspec((H * D, AV)), cspec((AV,)),
                  cspec((H * D, AV)), cspec((AV,)),
                  cspec((AV,)), cspec((AV,)),
                  cspec((H * D, OUT)), cspec((OUT,)),
                  cspec((H * D, OUT)), cspec((OUT,))],
        out_specs=(pl.BlockSpec((BT, OUT), lambda p, i: (i, 0)),
                   pl.BlockSpec((BT, OUT), lambda p, i: (i, 0)),
                   pl.BlockSpec((BT, H * D), lambda p, i: (i, 0)),
                   pl.BlockSpec((BT, H * D), lambda p, i: (i, 0))),
        out_shape=(
            jax.ShapeDtypeStruct((T, OUT), f32),
            jax.ShapeDtypeStruct((T, OUT), f32),
            jax.ShapeDtypeStruct((T, H * D), f32),
            jax.ShapeDtypeStruct((T, H * D), f32),
        ),
        scratch_shapes=[pltpu.VMEM((4, AV), f32),
                        pltpu.SMEM((4,), f32)],
    )(o_raw, o_raw, o_raw, o_raw,
      fc1_u_W, fc1_u_b, fc1_i_W, fc1_i_b, fc2_u_W, fc2_i_W,
      fc_user_W, fc_user_b, fc_item_W, fc_item_b)
    return (lu, li, hu, hi)
